# baseline (speedup 1.0000x reference)
import functools

import jax
import jax.numpy as jnp
from jax import lax
from jax.experimental import pallas as pl
from jax.experimental.pallas import tpu as pltpu
from jax.experimental.pallas import tpu_sc as plsc

EXPERT_DIM = 512
NUM_EXPERTS = 246
BATCH = 16384
B_SC = 8192
B_TC = BATCH - B_SC
BLK_TC = 1024
_NBLK_TC = B_TC // BLK_TC
_BLK0_TC = B_SC // BLK_TC

_info = plsc.get_sparse_core_info()
_NC, _NS = _info.num_cores, _info.num_subcores
_NW = _NC * _NS
_B_PER_W = B_SC // _NW
_CHUNK = 32
_NB = 4
_NCHUNK = _B_PER_W // _CHUNK
_LOOKAHEAD = 2


def _transform_body(table_ref, w_ref, b_ref, out_ref):
    t = lax.dot_general(
        table_ref[...], w_ref[...],
        dimension_numbers=(((1,), (1,)), ((), ())),
        preferred_element_type=jnp.float32,
    )
    out_ref[...] = t + b_ref[...]


def _transform_table(table, W, b2d):
    return pl.pallas_call(
        _transform_body,
        out_shape=jax.ShapeDtypeStruct((NUM_EXPERTS, EXPERT_DIM), jnp.float32),
    )(table, W, b2d)


@functools.partial(
    pl.kernel,
    out_type=jax.ShapeDtypeStruct((BATCH, EXPERT_DIM), jnp.float32),
    mesh=plsc.VectorSubcoreMesh(core_axis_name="c", subcore_axis_name="s"),
    scratch_types=[
        pltpu.VMEM((_B_PER_W,), jnp.int32),
        pltpu.VMEM((_NB, _CHUNK, EXPERT_DIM), jnp.float32),
        pltpu.SemaphoreType.DMA((_NB,)),
        pltpu.SemaphoreType.DMA((_NB,)),
    ],
)
def _gather_rows(t_hbm, idx_hbm, out_hbm, idx_v, rows, gsem, ssem):
    wid = lax.axis_index("s") * _NC + lax.axis_index("c")
    base = wid * _B_PER_W
    pltpu.sync_copy(idx_hbm.at[pl.ds(base, _B_PER_W)], idx_v)

    def gather(j):
        s = j % _NB
        return pltpu.async_copy(
            t_hbm.at[idx_v.at[pl.ds(j * _CHUNK, _CHUNK)]], rows.at[s], gsem.at[s]
        )

    def scatter(c):
        s = c % _NB
        return pltpu.async_copy(
            rows.at[s], out_hbm.at[pl.ds(base + c * _CHUNK, _CHUNK)], ssem.at[s]
        )

    gcops = [None] * _NCHUNK
    scops = [None] * _NCHUNK
    for j in range(_LOOKAHEAD):
        gcops[j] = gather(j)
    for c in range(_NCHUNK):
        j = c + _LOOKAHEAD
        if j < _NCHUNK:
            if j >= _NB:
                scops[j - _NB].wait()
            gcops[j] = gather(j)
        gcops[c].wait()
        scops[c] = scatter(c)
    for c in range(_NCHUNK - _NB, _NCHUNK):
        scops[c].wait()


def _onehot_body(idx_ref, t_ref, out_ref):
    ids = idx_ref[0, 0, :]
    oh = (ids[:, None]
          == lax.broadcasted_iota(jnp.int32, (BLK_TC, NUM_EXPERTS), 1)
          ).astype(jnp.float32)
    out_ref[...] = jnp.dot(oh, t_ref[...], preferred_element_type=jnp.float32)


def _onehot_compute(idx3d, t):
    return pl.pallas_call(
        _onehot_body,
        grid=(_NBLK_TC,),
        in_specs=[
            pl.BlockSpec((1, 1, BLK_TC), lambda i: (_BLK0_TC + i, 0, 0)),
            pl.BlockSpec((NUM_EXPERTS, EXPERT_DIM), lambda i: (0, 0)),
        ],
        out_specs=pl.BlockSpec((BLK_TC, EXPERT_DIM), lambda i: (i, 0)),
        out_shape=jax.ShapeDtypeStruct((B_TC, EXPERT_DIM), jnp.float32),
    )(idx3d, t)


def kernel(expert_id, table, W, b):
    t = _transform_table(table, W, b.reshape(1, EXPERT_DIM))
    idx = expert_id.astype(jnp.int32)
    y = _gather_rows(t, idx)
    idx3d = idx.reshape(BATCH // BLK_TC, 1, BLK_TC)
    z = _onehot_compute(idx3d, t)
    return lax.dynamic_update_slice(y, z, (B_SC, 0))


# speedup vs baseline: 1.1639x; 1.1639x over previous
import functools

import jax
import jax.numpy as jnp
from jax import lax
from jax.experimental import pallas as pl
from jax.experimental.pallas import tpu as pltpu
from jax.experimental.pallas import tpu_sc as plsc

EXPERT_DIM = 512
NUM_EXPERTS = 246
BATCH = 16384
B_SC = 8192
B_TC = BATCH - B_SC
BLK_TC = 1024
_NBLK_TC = B_TC // BLK_TC
_BLK0_TC = B_SC // BLK_TC

_info = plsc.get_sparse_core_info()
_NC, _NS = _info.num_cores, _info.num_subcores
_NW = _NC * _NS
_B_PER_W = B_SC // _NW
_CHUNK = 32
_NB = 4
_NCHUNK = _B_PER_W // _CHUNK
_LOOKAHEAD = 2


def _transform_body(table_ref, w_ref, b_ref, out_ref):
    t = lax.dot_general(
        table_ref[...], w_ref[...],
        dimension_numbers=(((1,), (1,)), ((), ())),
        preferred_element_type=jnp.float32,
    )
    out_ref[...] = t + b_ref[...]


def _transform_table(table, W, b2d):
    return pl.pallas_call(
        _transform_body,
        out_shape=jax.ShapeDtypeStruct((NUM_EXPERTS, EXPERT_DIM), jnp.float32),
    )(table, W, b2d)


@functools.partial(
    pl.kernel,
    out_type=jax.ShapeDtypeStruct((BATCH, EXPERT_DIM), jnp.float32),
    mesh=plsc.VectorSubcoreMesh(core_axis_name="c", subcore_axis_name="s"),
    scratch_types=[
        pltpu.VMEM((_B_PER_W,), jnp.int32),
        pltpu.VMEM((_NB, _CHUNK, EXPERT_DIM), jnp.float32),
        pltpu.SemaphoreType.DMA((_NB,)),
        pltpu.SemaphoreType.DMA((_NB,)),
    ],
)
def _gather_rows(t_hbm, idx_hbm, out_hbm, idx_v, rows, gsem, ssem):
    wid = lax.axis_index("s") * _NC + lax.axis_index("c")
    base = wid * _B_PER_W
    pltpu.sync_copy(idx_hbm.at[pl.ds(base, _B_PER_W)], idx_v)

    def gather(j):
        s = j % _NB
        return pltpu.async_copy(
            t_hbm.at[idx_v.at[pl.ds(j * _CHUNK, _CHUNK)]], rows.at[s], gsem.at[s]
        )

    def scatter(c):
        s = c % _NB
        return pltpu.async_copy(
            rows.at[s], out_hbm.at[pl.ds(base + c * _CHUNK, _CHUNK)], ssem.at[s]
        )

    gcops = [None] * _NCHUNK
    scops = [None] * _NCHUNK
    for j in range(_LOOKAHEAD):
        gcops[j] = gather(j)
    for c in range(_NCHUNK):
        j = c + _LOOKAHEAD
        if j < _NCHUNK:
            if j >= _NB:
                scops[j - _NB].wait()
            gcops[j] = gather(j)
        gcops[c].wait()
        scops[c] = scatter(c)
    for c in range(_NCHUNK - _NB, _NCHUNK):
        scops[c].wait()


def _onehot_body(idx_ref, t_ref, y_ref, out_ref):
    del y_ref
    ids = idx_ref[0, 0, :]
    oh = (ids[:, None]
          == lax.broadcasted_iota(jnp.int32, (BLK_TC, NUM_EXPERTS), 1)
          ).astype(jnp.float32)
    out_ref[...] = jnp.dot(oh, t_ref[...], preferred_element_type=jnp.float32)


def _onehot_fill(idx3d, t, y):
    return pl.pallas_call(
        _onehot_body,
        grid=(_NBLK_TC,),
        in_specs=[
            pl.BlockSpec((1, 1, BLK_TC), lambda i: (_BLK0_TC + i, 0, 0)),
            pl.BlockSpec((NUM_EXPERTS, EXPERT_DIM), lambda i: (0, 0)),
            pl.BlockSpec(memory_space=pl.ANY),
        ],
        out_specs=pl.BlockSpec(
            (BLK_TC, EXPERT_DIM), lambda i: (_BLK0_TC + i, 0)
        ),
        out_shape=jax.ShapeDtypeStruct((BATCH, EXPERT_DIM), jnp.float32),
        input_output_aliases={2: 0},
    )(idx3d, t, y)


def kernel(expert_id, table, W, b):
    t = _transform_table(table, W, b.reshape(1, EXPERT_DIM))
    idx = expert_id.astype(jnp.int32)
    y = _gather_rows(t, idx)
    idx3d = idx.reshape(BATCH // BLK_TC, 1, BLK_TC)
    return _onehot_fill(idx3d, t, y)


# speedup vs baseline: 1.2477x; 1.0720x over previous
import functools

import jax
import jax.numpy as jnp
from jax import lax
from jax.experimental import pallas as pl
from jax.experimental.pallas import tpu as pltpu
from jax.experimental.pallas import tpu_sc as plsc

EXPERT_DIM = 512
NUM_EXPERTS = 246
BATCH = 16384
B_SC = 6144
B_TC = BATCH - B_SC
BLK_TC = 1024
_NBLK_TC = B_TC // BLK_TC
_BLK0_TC = B_SC // BLK_TC

_info = plsc.get_sparse_core_info()
_NC, _NS = _info.num_cores, _info.num_subcores
_NW = _NC * _NS
_B_PER_W = B_SC // _NW
_CHUNK = 32
_NB = 4
_NCHUNK = _B_PER_W // _CHUNK
_LOOKAHEAD = 2


def _transform_body(table_ref, w_ref, b_ref, out_ref):
    t = lax.dot_general(
        table_ref[...], w_ref[...],
        dimension_numbers=(((1,), (1,)), ((), ())),
        preferred_element_type=jnp.float32,
    )
    out_ref[...] = t + b_ref[...]


def _transform_table(table, W, b2d):
    return pl.pallas_call(
        _transform_body,
        out_shape=jax.ShapeDtypeStruct((NUM_EXPERTS, EXPERT_DIM), jnp.float32),
    )(table, W, b2d)


@functools.partial(
    pl.kernel,
    out_type=jax.ShapeDtypeStruct((BATCH, EXPERT_DIM), jnp.float32),
    mesh=plsc.VectorSubcoreMesh(core_axis_name="c", subcore_axis_name="s"),
    scratch_types=[
        pltpu.VMEM((_B_PER_W,), jnp.int32),
        pltpu.VMEM((_NB, _CHUNK, EXPERT_DIM), jnp.float32),
        pltpu.SemaphoreType.DMA((_NB,)),
        pltpu.SemaphoreType.DMA((_NB,)),
    ],
)
def _gather_rows(t_hbm, idx_hbm, out_hbm, idx_v, rows, gsem, ssem):
    wid = lax.axis_index("s") * _NC + lax.axis_index("c")
    base = wid * _B_PER_W
    pltpu.sync_copy(idx_hbm.at[pl.ds(base, _B_PER_W)], idx_v)

    def gather(j):
        s = j % _NB
        return pltpu.async_copy(
            t_hbm.at[idx_v.at[pl.ds(j * _CHUNK, _CHUNK)]], rows.at[s], gsem.at[s]
        )

    def scatter(c):
        s = c % _NB
        return pltpu.async_copy(
            rows.at[s], out_hbm.at[pl.ds(base + c * _CHUNK, _CHUNK)], ssem.at[s]
        )

    gcops = [None] * _NCHUNK
    scops = [None] * _NCHUNK
    for j in range(_LOOKAHEAD):
        gcops[j] = gather(j)
    for c in range(_NCHUNK):
        j = c + _LOOKAHEAD
        if j < _NCHUNK:
            if j >= _NB:
                scops[j - _NB].wait()
            gcops[j] = gather(j)
        gcops[c].wait()
        scops[c] = scatter(c)
    for c in range(_NCHUNK - _NB, _NCHUNK):
        scops[c].wait()


def _onehot_body(idx_ref, t_ref, y_ref, out_ref):
    del y_ref
    ids = idx_ref[0, 0, :]
    oh = (ids[:, None]
          == lax.broadcasted_iota(jnp.int32, (BLK_TC, NUM_EXPERTS), 1)
          ).astype(jnp.float32)
    out_ref[...] = jnp.dot(oh, t_ref[...], preferred_element_type=jnp.float32)


def _onehot_fill(idx3d, t, y):
    return pl.pallas_call(
        _onehot_body,
        grid=(_NBLK_TC,),
        in_specs=[
            pl.BlockSpec((1, 1, BLK_TC), lambda i: (_BLK0_TC + i, 0, 0)),
            pl.BlockSpec((NUM_EXPERTS, EXPERT_DIM), lambda i: (0, 0)),
            pl.BlockSpec(memory_space=pl.ANY),
        ],
        out_specs=pl.BlockSpec(
            (BLK_TC, EXPERT_DIM), lambda i: (_BLK0_TC + i, 0)
        ),
        out_shape=jax.ShapeDtypeStruct((BATCH, EXPERT_DIM), jnp.float32),
        input_output_aliases={2: 0},
    )(idx3d, t, y)


def kernel(expert_id, table, W, b):
    t = _transform_table(table, W, b.reshape(1, EXPERT_DIM))
    idx = expert_id.astype(jnp.int32)
    y = _gather_rows(t, idx)
    idx3d = idx.reshape(BATCH // BLK_TC, 1, BLK_TC)
    return _onehot_fill(idx3d, t, y)


# speedup vs baseline: 1.3159x; 1.0546x over previous
import functools

import jax
import jax.numpy as jnp
from jax import lax
from jax.experimental import pallas as pl
from jax.experimental.pallas import tpu as pltpu
from jax.experimental.pallas import tpu_sc as plsc

EXPERT_DIM = 512
NUM_EXPERTS = 246
BATCH = 16384
B_SC = 6144
B_TC = BATCH - B_SC
BLK_TC = 2048
_NBLK_TC = B_TC // BLK_TC
_BLK0_TC = B_SC // BLK_TC

_info = plsc.get_sparse_core_info()
_NC, _NS = _info.num_cores, _info.num_subcores
_NW = _NC * _NS
_B_PER_W = B_SC // _NW
_CHUNK = 32
_NB = 4
_NCHUNK = _B_PER_W // _CHUNK
_LOOKAHEAD = 2


def _transform_body(table_ref, w_ref, b_ref, out_ref):
    t = lax.dot_general(
        table_ref[...], w_ref[...],
        dimension_numbers=(((1,), (1,)), ((), ())),
        preferred_element_type=jnp.float32,
    )
    out_ref[...] = t + b_ref[...]


def _transform_table(table, W, b2d):
    return pl.pallas_call(
        _transform_body,
        out_shape=jax.ShapeDtypeStruct((NUM_EXPERTS, EXPERT_DIM), jnp.float32),
    )(table, W, b2d)


@functools.partial(
    pl.kernel,
    out_type=jax.ShapeDtypeStruct((BATCH, EXPERT_DIM), jnp.float32),
    mesh=plsc.VectorSubcoreMesh(core_axis_name="c", subcore_axis_name="s"),
    scratch_types=[
        pltpu.VMEM((_B_PER_W,), jnp.int32),
        pltpu.VMEM((_NB, _CHUNK, EXPERT_DIM), jnp.float32),
        pltpu.SemaphoreType.DMA((_NB,)),
        pltpu.SemaphoreType.DMA((_NB,)),
    ],
)
def _gather_rows(t_hbm, idx_hbm, out_hbm, idx_v, rows, gsem, ssem):
    wid = lax.axis_index("s") * _NC + lax.axis_index("c")
    base = wid * _B_PER_W
    pltpu.sync_copy(idx_hbm.at[pl.ds(base, _B_PER_W)], idx_v)

    def gather(j):
        s = j % _NB
        return pltpu.async_copy(
            t_hbm.at[idx_v.at[pl.ds(j * _CHUNK, _CHUNK)]], rows.at[s], gsem.at[s]
        )

    def scatter(c):
        s = c % _NB
        return pltpu.async_copy(
            rows.at[s], out_hbm.at[pl.ds(base + c * _CHUNK, _CHUNK)], ssem.at[s]
        )

    gcops = [None] * _NCHUNK
    scops = [None] * _NCHUNK
    for j in range(_LOOKAHEAD):
        gcops[j] = gather(j)
    for c in range(_NCHUNK):
        j = c + _LOOKAHEAD
        if j < _NCHUNK:
            if j >= _NB:
                scops[j - _NB].wait()
            gcops[j] = gather(j)
        gcops[c].wait()
        scops[c] = scatter(c)
    for c in range(_NCHUNK - _NB, _NCHUNK):
        scops[c].wait()


def _onehot_body(idx_ref, t_ref, y_ref, out_ref):
    del y_ref
    ids = idx_ref[0, 0, :]
    oh = (ids[:, None]
          == lax.broadcasted_iota(jnp.int32, (BLK_TC, NUM_EXPERTS), 1)
          ).astype(jnp.float32)
    out_ref[...] = jnp.dot(oh, t_ref[...], preferred_element_type=jnp.float32)


def _onehot_fill(idx3d, t, y):
    return pl.pallas_call(
        _onehot_body,
        grid=(_NBLK_TC,),
        in_specs=[
            pl.BlockSpec((1, 1, BLK_TC), lambda i: (_BLK0_TC + i, 0, 0)),
            pl.BlockSpec((NUM_EXPERTS, EXPERT_DIM), lambda i: (0, 0)),
            pl.BlockSpec(memory_space=pl.ANY),
        ],
        out_specs=pl.BlockSpec(
            (BLK_TC, EXPERT_DIM), lambda i: (_BLK0_TC + i, 0)
        ),
        out_shape=jax.ShapeDtypeStruct((BATCH, EXPERT_DIM), jnp.float32),
        input_output_aliases={2: 0},
    )(idx3d, t, y)


def kernel(expert_id, table, W, b):
    t = _transform_table(table, W, b.reshape(1, EXPERT_DIM))
    idx = expert_id.astype(jnp.int32)
    y = _gather_rows(t, idx)
    idx3d = idx.reshape(BATCH // BLK_TC, 1, BLK_TC)
    return _onehot_fill(idx3d, t, y)
